# baseline (speedup 1.0000x reference)
import functools

import jax
import jax.numpy as jnp
import numpy as np
from jax import lax
from jax.experimental import pallas as pl
from jax.experimental.pallas import tpu as pltpu
from jax.experimental.pallas import tpu_sc as plsc

D = 768
DH = 3072
D_INNER = 1536
S = 2048
TOP_K = 2
N_HEADS = 8
DHEAD = 96
CHUNK = 256
MTILE = 128
NT = 38


def _gather_rows(table, idx):
    B = idx.shape[0]
    Dt = table.shape[1]
    dt = table.dtype
    NW = 32
    bpw = B // NW
    mesh = plsc.VectorSubcoreMesh(core_axis_name="c", subcore_axis_name="s")

    @functools.partial(
        pl.kernel, mesh=mesh,
        out_type=jax.ShapeDtypeStruct((B, Dt), dt),
        scratch_types=[
            pltpu.VMEM((bpw,), jnp.int32),
            pltpu.VMEM((bpw, Dt), dt),
            pltpu.SemaphoreType.DMA,
        ],
    )
    def k(table_hbm, idx_hbm, out_hbm, idx_v, rows_v, sem):
        wid = lax.axis_index("s") * 2 + lax.axis_index("c")
        base = wid * bpw
        pltpu.sync_copy(idx_hbm.at[pl.ds(base, bpw)], idx_v)
        csz = 8
        copies = [
            pltpu.async_copy(
                table_hbm.at[idx_v.at[pl.ds(ci * csz, csz)]],
                rows_v.at[pl.ds(ci * csz, csz)], sem)
            for ci in range(bpw // csz)
        ]
        for c in copies:
            c.wait()
        pltpu.sync_copy(rows_v, out_hbm.at[pl.ds(base, bpw)])

    return k(table, idx)


def _pack2(x):
    c = x.shape[1] // 2
    u = lax.bitcast_convert_type(x, jnp.int32)
    rne = u + 0x7FFF + (lax.shift_right_logical(u, 16) & 1)
    lo = lax.shift_right_logical(rne[:, :c], 16)
    hi = rne[:, c:] & jnp.int32(-65536)
    return lo | hi


def _unpack2(xi):
    lo = lax.bitcast_convert_type(lax.shift_left(xi, 16), jnp.float32)
    hi = lax.bitcast_convert_type(xi & jnp.int32(-65536), jnp.float32)
    return jnp.concatenate([lo, hi], axis=1)


def _mamba_body(x_ref, win_ref, bin_ref, cw_ref, cb_ref, alog_ref,
                wout_ref, bout_ref, o_ref, op_ref,
                uprev_ref, apow_ref, carry_ref):
    i = pl.program_id(0)
    xz = jnp.dot(x_ref[...].astype(jnp.bfloat16), win_ref[...],
                 preferred_element_type=jnp.float32) + bin_ref[...]
    u = xz[:, :D_INNER]
    z = xz[:, D_INNER:]

    a = jax.nn.sigmoid(alog_ref[...])

    @pl.when(i == 0)
    def _init():
        uprev_ref[...] = jnp.zeros_like(uprev_ref)
        carry_ref[...] = jnp.zeros_like(carry_ref)
        t = lax.broadcasted_iota(jnp.int32, (CHUNK, D_INNER), 0).astype(
            jnp.float32)
        apow_ref[...] = jnp.exp(jnp.log(a) * (t + 1.0))

    ext = jnp.concatenate([uprev_ref[5:8, :], u], axis=0)
    y = (ext[0:CHUNK, :] * cw_ref[0:1, :]
         + ext[1:CHUNK + 1, :] * cw_ref[1:2, :]
         + ext[2:CHUNK + 2, :] * cw_ref[2:3, :]
         + ext[3:CHUNK + 3, :] * cw_ref[3:4, :]) + cb_ref[...]
    uprev_ref[5:8, :] = u[CHUNK - 3:CHUNK, :]
    v = jax.nn.silu(y)

    h = v
    ak = a
    k = 1
    while k < CHUNK:
        top = jnp.zeros((k, D_INNER), jnp.float32)
        h = h + ak * jnp.concatenate([top, h[:CHUNK - k, :]], axis=0)
        ak = ak * ak
        k *= 2
    h = h + apow_ref[...] * carry_ref[...]
    carry_ref[...] = h[CHUNK - 1:CHUNK, :]

    g = h * jax.nn.silu(z)
    m = jnp.dot(g.astype(jnp.bfloat16), wout_ref[...],
                preferred_element_type=jnp.float32) + bout_ref[...]
    o_ref[...] = m
    op_ref[...] = _pack2(m)


def _mamba(x, mp):
    win = mp["win"].astype(jnp.bfloat16)
    bin_ = mp["bin"].reshape(1, 2 * D_INNER)
    cw = mp["conv_w"].T
    cb = mp["conv_b"].reshape(1, D_INNER)
    alog = mp["a_logit"].reshape(1, D_INNER)
    wout = mp["wout"].astype(jnp.bfloat16)
    bout = mp["bout"].reshape(1, D)
    n = S // CHUNK
    return pl.pallas_call(
        _mamba_body,
        grid=(n,),
        in_specs=[
            pl.BlockSpec((CHUNK, D), lambda i: (i, 0)),
            pl.BlockSpec((D, 2 * D_INNER), lambda i: (0, 0)),
            pl.BlockSpec((1, 2 * D_INNER), lambda i: (0, 0)),
            pl.BlockSpec((4, D_INNER), lambda i: (0, 0)),
            pl.BlockSpec((1, D_INNER), lambda i: (0, 0)),
            pl.BlockSpec((1, D_INNER), lambda i: (0, 0)),
            pl.BlockSpec((D_INNER, D), lambda i: (0, 0)),
            pl.BlockSpec((1, D), lambda i: (0, 0)),
        ],
        out_specs=[pl.BlockSpec((CHUNK, D), lambda i: (i, 0)),
                   pl.BlockSpec((CHUNK, D // 2), lambda i: (i, 0))],
        out_shape=[jax.ShapeDtypeStruct((S, D), jnp.float32),
                   jax.ShapeDtypeStruct((S, D // 2), jnp.int32)],
        scratch_shapes=[
            pltpu.VMEM((8, D_INNER), jnp.float32),
            pltpu.VMEM((CHUNK, D_INNER), jnp.float32),
            pltpu.VMEM((1, D_INNER), jnp.float32),
        ],
        interpret=False,
    )(x, win, bin_, cw, cb, alog, wout, bout)


def _router_body(ne, x_ref, w_ref, b_ref, wv_ref, ii_ref):
    E = ne + 1
    logits = jnp.dot(x_ref[...], w_ref[...],
                     preferred_element_type=jnp.float32) + b_ref[...]
    lane = lax.broadcasted_iota(jnp.int32, (S, 128), 1)
    valid = lane < E
    lm = jnp.where(valid, logits, -1e30)
    mx = jnp.max(lm, axis=1, keepdims=True)
    e = jnp.where(valid, jnp.exp(lm - mx), 0.0)
    p = e / jnp.sum(e, axis=1, keepdims=True)
    m1 = jnp.max(p, axis=1, keepdims=True)
    i1 = jnp.min(jnp.where(p == m1, lane, 127), axis=1, keepdims=True)
    p2 = jnp.where(lane == i1, -1.0, p)
    m2 = jnp.max(p2, axis=1, keepdims=True)
    i2 = jnp.min(jnp.where(p2 == m2, lane, 127), axis=1, keepdims=True)
    z6 = jnp.zeros((S, 6), jnp.float32)
    wv_ref[...] = jnp.concatenate([m1, m2, z6], axis=1)
    ii_ref[...] = jnp.concatenate(
        [i1, i2, jnp.zeros((S, 6), jnp.int32)], axis=1)


def _router(x, rp, ne):
    E = ne + 1
    w = jnp.pad(rp["w"], ((0, 0), (0, 128 - E)))
    b = jnp.pad(rp["b"], (0, 128 - E)).reshape(1, 128)
    return pl.pallas_call(
        functools.partial(_router_body, ne),
        out_shape=[jax.ShapeDtypeStruct((S, 8), jnp.float32),
                   jax.ShapeDtypeStruct((S, 8), jnp.int32)],
        interpret=False,
    )(x, w, b)


def _route_plan(ii, ne):
    e_flat = ii[:, :TOP_K].reshape(S * TOP_K)
    enz = jnp.minimum(e_flat, ne)
    perm = jnp.argsort(enz, stable=True).astype(jnp.int32)
    ip = jnp.zeros((S * TOP_K,), jnp.int32).at[perm].set(
        jnp.arange(S * TOP_K, dtype=jnp.int32))
    counts = jnp.sum(enz[:, None] == jnp.arange(ne)[None, :],
                     axis=0).astype(jnp.int32)
    offs = jnp.concatenate([jnp.zeros((1,), jnp.int32),
                            jnp.cumsum(counts)])[:ne]
    tiles = (counts + (MTILE - 1)) // MTILE
    tcum = jnp.concatenate([jnp.zeros((1,), jnp.int32), jnp.cumsum(tiles)])
    t_ar = jnp.arange(NT, dtype=jnp.int32)
    te = jnp.minimum(jnp.sum(t_ar[:, None] >= tcum[None, 1:], axis=1),
                     ne - 1).astype(jnp.int32)
    pbase = tcum[:ne] * MTILE

    r = jnp.arange(NT * MTILE, dtype=jnp.int32)
    e_r = te[r // MTILE]
    within = r - pbase[e_r]
    valid = within < counts[e_r]
    src = perm[jnp.clip(offs[e_r] + within, 0, S * TOP_K - 1)]
    disp_idx = jnp.where(valid, src // TOP_K, 0).astype(jnp.int32)

    ej = jnp.minimum(enz, ne - 1)
    padpos = jnp.where(enz < ne, pbase[ej] + (ip - offs[ej]), 0)
    comb_idx = padpos.reshape(S, TOP_K).T.reshape(S * TOP_K).astype(jnp.int32)
    return disp_idx, comb_idx, te


def _expA_body(te_ref, tf_ref, xg_ref, w1_ref, b1_ref, h_ref):
    t = pl.program_id(0)
    xg = _unpack2(xg_ref[...]).astype(jnp.bfloat16)
    hh = jnp.dot(xg, w1_ref[0],
                 preferred_element_type=jnp.float32) + b1_ref[0]
    p = hh[:, :DH]
    q = hh[:, DH:]
    flag = tf_ref[te_ref[t]]
    h = jnp.where(flag == 0, jax.nn.silu(p) * q, p * jax.nn.gelu(q))
    h_ref[...] = _pack2(h)


def _expB_body(te_ref, tf_ref, h_ref, w2_ref, b2_ref, y_ref):
    h = _unpack2(h_ref[...]).astype(jnp.bfloat16)
    y = jnp.dot(h, w2_ref[0],
                preferred_element_type=jnp.float32) + b2_ref[0]
    y_ref[...] = _pack2(y)


def _experts(xg, W1, B1, W2, B2, tflags, te):
    ne = W1.shape[0]
    h = pl.pallas_call(
        _expA_body,
        grid_spec=pltpu.PrefetchScalarGridSpec(
            num_scalar_prefetch=2,
            grid=(NT,),
            in_specs=[
                pl.BlockSpec((MTILE, D // 2), lambda t, te, tf: (t, 0)),
                pl.BlockSpec((1, D, 2 * DH), lambda t, te, tf: (te[t], 0, 0)),
                pl.BlockSpec((1, 1, 2 * DH), lambda t, te, tf: (te[t], 0, 0)),
            ],
            out_specs=pl.BlockSpec((MTILE, DH // 2), lambda t, te, tf: (t, 0)),
        ),
        out_shape=jax.ShapeDtypeStruct((NT * MTILE, DH // 2), jnp.int32),
        interpret=False,
    )(te, tflags, xg, W1, B1.reshape(ne, 1, 2 * DH))
    y = pl.pallas_call(
        _expB_body,
        grid_spec=pltpu.PrefetchScalarGridSpec(
            num_scalar_prefetch=2,
            grid=(NT,),
            in_specs=[
                pl.BlockSpec((MTILE, DH // 2), lambda t, te, tf: (t, 0)),
                pl.BlockSpec((1, DH, D), lambda t, te, tf: (te[t], 0, 0)),
                pl.BlockSpec((1, 1, D), lambda t, te, tf: (te[t], 0, 0)),
            ],
            out_specs=pl.BlockSpec((MTILE, D // 2), lambda t, te, tf: (t, 0)),
        ),
        out_shape=jax.ShapeDtypeStruct((NT * MTILE, D // 2), jnp.int32),
        interpret=False,
    )(te, tflags, h, W2, B2.reshape(ne, 1, D))
    return y


def _ln(x, g, b):
    m = jnp.mean(x, axis=-1, keepdims=True)
    v = jnp.mean((x - m) ** 2, axis=-1, keepdims=True)
    return (x - m) / jnp.sqrt(v + 1e-5) * g + b


def _moe_sum(ne, g_ref, m_ref, wv_ref, ii_ref):
    g0 = _unpack2(g_ref[0:S, :])
    g1 = _unpack2(g_ref[S:2 * S, :])
    w0 = wv_ref[:, 0:1]
    w1 = wv_ref[:, 1:2]
    i0 = ii_ref[:, 0:1]
    i1 = ii_ref[:, 1:2]
    nid0 = (i0 != ne).astype(jnp.float32)
    nid1 = (i1 != ne).astype(jnp.float32)
    idw = w0 * (1.0 - nid0) + w1 * (1.0 - nid1)
    return w0 * nid0 * g0 + w1 * nid1 * g1 + idw * m_ref[...]


def _combine_body(ne, g_ref, m_ref, wv_ref, ii_ref, o_ref):
    o_ref[...] = _moe_sum(ne, g_ref, m_ref, wv_ref, ii_ref)


def _combine_ln_body(ne, g_ref, m_ref, wv_ref, ii_ref, res_ref,
                     lg_ref, lb_ref, o_ref):
    moe = _moe_sum(ne, g_ref, m_ref, wv_ref, ii_ref)
    o_ref[...] = _ln(moe + res_ref[...], lg_ref[...], lb_ref[...])


def _combine(g, m, wv, ii, ne):
    return pl.pallas_call(
        functools.partial(_combine_body, ne),
        out_shape=jax.ShapeDtypeStruct((S, D), jnp.float32),
        interpret=False,
    )(g, m, wv, ii)


def _combine_ln(g, m, wv, ii, res, lnp, ne):
    return pl.pallas_call(
        functools.partial(_combine_ln_body, ne),
        out_shape=jax.ShapeDtypeStruct((S, D), jnp.float32),
        interpret=False,
    )(g, m, wv, ii, res, lnp["g"].reshape(1, D), lnp["b"].reshape(1, D))


def _attn_body(x_ref, wq_ref, wk_ref, wv_ref, bq_ref, bk_ref, bv_ref,
               wo_ref, bo_ref, o_ref):
    h = pl.program_id(0)
    x = x_ref[...].astype(jnp.bfloat16)
    q = jnp.dot(x, wq_ref[0], preferred_element_type=jnp.float32) + bq_ref[0]
    k = jnp.dot(x, wk_ref[0], preferred_element_type=jnp.float32) + bk_ref[0]
    v = jnp.dot(x, wv_ref[0], preferred_element_type=jnp.float32) + bv_ref[0]
    s = lax.dot_general(q.astype(jnp.bfloat16), k.astype(jnp.bfloat16),
                        (((1,), (1,)), ((), ())),
                        preferred_element_type=jnp.float32)
    s = s * np.float32(1.0 / np.sqrt(DHEAD))
    mx = jnp.max(s, axis=1, keepdims=True)
    es = jnp.exp(s - mx)
    att = es / jnp.sum(es, axis=1, keepdims=True)
    ov = jnp.dot(att.astype(jnp.bfloat16), v.astype(jnp.bfloat16),
                 preferred_element_type=jnp.float32)
    contrib = jnp.dot(ov.astype(jnp.bfloat16), wo_ref[0],
                      preferred_element_type=jnp.float32)

    @pl.when(h == 0)
    def _init():
        o_ref[...] = jnp.broadcast_to(bo_ref[...], (S, D))

    o_ref[...] += contrib


def _attn(x, ap):
    wqkv = ap["wqkv"]
    bqkv = ap["bqkv"]

    def head_w(wpart):
        w = wpart.reshape(D, N_HEADS, DHEAD).transpose(1, 0, 2)
        return jnp.pad(w, ((0, 0), (0, 0), (0, 128 - DHEAD))).astype(
            jnp.bfloat16)

    def head_b(bpart):
        b = bpart.reshape(N_HEADS, 1, DHEAD)
        return jnp.pad(b, ((0, 0), (0, 0), (0, 128 - DHEAD)))

    wq = head_w(wqkv[:, :D])
    wk = head_w(wqkv[:, D:2 * D])
    wv = head_w(wqkv[:, 2 * D:])
    bq = head_b(bqkv[:D])
    bk = head_b(bqkv[D:2 * D])
    bv = head_b(bqkv[2 * D:])
    wo = jnp.pad(ap["wo"].reshape(N_HEADS, DHEAD, D),
                 ((0, 0), (0, 128 - DHEAD), (0, 0))).astype(jnp.bfloat16)
    bo = ap["bo"].reshape(1, D)
    return pl.pallas_call(
        _attn_body,
        grid=(N_HEADS,),
        in_specs=[
            pl.BlockSpec((S, D), lambda h: (0, 0)),
            pl.BlockSpec((1, D, 128), lambda h: (h, 0, 0)),
            pl.BlockSpec((1, D, 128), lambda h: (h, 0, 0)),
            pl.BlockSpec((1, D, 128), lambda h: (h, 0, 0)),
            pl.BlockSpec((1, 1, 128), lambda h: (h, 0, 0)),
            pl.BlockSpec((1, 1, 128), lambda h: (h, 0, 0)),
            pl.BlockSpec((1, 1, 128), lambda h: (h, 0, 0)),
            pl.BlockSpec((1, 128, D), lambda h: (h, 0, 0)),
            pl.BlockSpec((1, D), lambda h: (0, 0)),
        ],
        out_specs=pl.BlockSpec((S, D), lambda h: (0, 0)),
        out_shape=jax.ShapeDtypeStruct((S, D), jnp.float32),
        interpret=False,
    )(x, wq, wk, wv, bq, bk, bv, wo, bo)


def _postattn_body(xm_ref, ao_ref, res_ref, ga_ref, ba_ref, g_ref, b_ref,
                   o_ref):
    y1 = _ln(xm_ref[...] + ao_ref[...], ga_ref[...], ba_ref[...])
    o_ref[...] = _ln(y1 + res_ref[...], g_ref[...], b_ref[...])


def _postattn(xm, ao, res, lna, lnp):
    return pl.pallas_call(
        _postattn_body,
        out_shape=jax.ShapeDtypeStruct((S, D), jnp.float32),
        interpret=False,
    )(xm, ao, res, lna["g"].reshape(1, D), lna["b"].reshape(1, D),
      lnp["g"].reshape(1, D), lnp["b"].reshape(1, D))


def _lmhead_body(x_ref, w_ref, o_ref):
    o_ref[...] = lax.dot_general(
        x_ref[...].astype(jnp.bfloat16), w_ref[...],
        (((1,), (1,)), ((), ())), preferred_element_type=jnp.float32)


def _lmhead(x, lm):
    V = lm.shape[0]
    VT = 1280
    wb = lm.astype(jnp.bfloat16)
    return pl.pallas_call(
        _lmhead_body,
        grid=(V // VT,),
        in_specs=[
            pl.BlockSpec((S, D), lambda j: (0, 0)),
            pl.BlockSpec((VT, D), lambda j: (j, 0)),
        ],
        out_specs=pl.BlockSpec((S, VT), lambda j: (0, j)),
        out_shape=jax.ShapeDtypeStruct((S, V), jnp.float32),
        interpret=False,
    )(x, wb)


_ETYPES = [["conv"] * 4 + ["geglu"] * 2 + ["id"],
           ["conv"] * 3 + ["geglu"] * 3 + ["id"],
           ["conv"] * 1 + ["geglu"] * 4 + ["id"]]
_NBLOCKS = [1, 2, 1]
_USE_ATTN = [False, False, True]


def _stack_experts(experts, etypes):
    w1s, b1s, w2s, b2s, flags = [], [], [], [], []
    for et, ep in zip(etypes, experts):
        if et == "conv":
            w1s.append(jnp.concatenate([ep["w1"], ep["w3"]], axis=1))
            b1s.append(jnp.concatenate([ep["b1"], ep["b3"]]))
            flags.append(0)
        elif et == "geglu":
            w1s.append(ep["w1"])
            b1s.append(ep["b1"])
            flags.append(1)
        else:
            continue
        w2s.append(ep["w2"])
        b2s.append(ep["b2"])
    return (jnp.stack(w1s).astype(jnp.bfloat16), jnp.stack(b1s),
            jnp.stack(w2s).astype(jnp.bfloat16), jnp.stack(b2s),
            jnp.asarray(flags, jnp.int32))


def kernel(input_ids, params):
    ids = input_ids.reshape(S).astype(jnp.int32)
    x = _gather_rows(params["embed"], ids)
    for s in range(3):
        st = params["stages"][s]
        etypes = _ETYPES[s]
        ne = len(etypes) - 1
        W1, B1, W2, B2, tflags = _stack_experts(st["experts"], etypes)
        for bi in range(_NBLOCKS[s]):
            bp = st["blocks"][bi]
            res = x
            m, mp = _mamba(x, bp["mamba"])
            wv, ii = _router(m, bp["router"], ne)
            disp_idx, comb_idx, te = _route_plan(ii, ne)
            xg = _gather_rows(mp, disp_idx)
            ys = _experts(xg, W1, B1, W2, B2, tflags, te)
            g = _gather_rows(ys, comb_idx)
            if _USE_ATTN[s]:
                xm = _combine(g, m, wv, ii, ne)
                ao = _attn(xm, bp["attn"])
                x = _postattn(xm, ao, res, bp["ln_attn"], bp["ln"])
            else:
                x = _combine_ln(g, m, wv, ii, res, bp["ln"], ne)
    logits = _lmhead(x, params["lm_head"])
    return logits.reshape(1, S, -1)


# speedup vs baseline: 1.2804x; 1.2804x over previous
import functools

import jax
import jax.numpy as jnp
import numpy as np
from jax import lax
from jax.experimental import pallas as pl
from jax.experimental.pallas import tpu as pltpu
from jax.experimental.pallas import tpu_sc as plsc

D = 768
DH = 3072
D_INNER = 1536
S = 2048
TOP_K = 2
N_HEADS = 8
DHEAD = 96
CHUNK = 256
MTILE = 256
NT = 22


def _gather_rows(table, idx):
    B = idx.shape[0]
    Dt = table.shape[1]
    dt = table.dtype
    NW = 32
    bpw = B // NW
    mesh = plsc.VectorSubcoreMesh(core_axis_name="c", subcore_axis_name="s")

    @functools.partial(
        pl.kernel, mesh=mesh,
        out_type=jax.ShapeDtypeStruct((B, Dt), dt),
        scratch_types=[
            pltpu.VMEM((bpw,), jnp.int32),
            pltpu.VMEM((bpw, Dt), dt),
            pltpu.SemaphoreType.DMA,
        ],
    )
    def k(table_hbm, idx_hbm, out_hbm, idx_v, rows_v, sem):
        wid = lax.axis_index("s") * 2 + lax.axis_index("c")
        base = wid * bpw
        pltpu.sync_copy(idx_hbm.at[pl.ds(base, bpw)], idx_v)
        csz = 8
        copies = [
            pltpu.async_copy(
                table_hbm.at[idx_v.at[pl.ds(ci * csz, csz)]],
                rows_v.at[pl.ds(ci * csz, csz)], sem)
            for ci in range(bpw // csz)
        ]
        for c in copies:
            c.wait()
        pltpu.sync_copy(rows_v, out_hbm.at[pl.ds(base, bpw)])

    return k(table, idx)


def _pack2(x):
    c = x.shape[1] // 2
    u = lax.bitcast_convert_type(x, jnp.int32)
    rne = u + 0x7FFF + (lax.shift_right_logical(u, 16) & 1)
    lo = lax.shift_right_logical(rne[:, :c], 16)
    hi = rne[:, c:] & jnp.int32(-65536)
    return lo | hi


def _unpack2(xi):
    lo = lax.bitcast_convert_type(lax.shift_left(xi, 16), jnp.float32)
    hi = lax.bitcast_convert_type(xi & jnp.int32(-65536), jnp.float32)
    return jnp.concatenate([lo, hi], axis=1)


def _mamba_body(x_ref, win_ref, bin_ref, cw_ref, cb_ref, alog_ref,
                wout_ref, bout_ref, o_ref, op_ref,
                uprev_ref, apow_ref, carry_ref):
    i = pl.program_id(0)
    xz = jnp.dot(x_ref[...].astype(jnp.bfloat16), win_ref[...],
                 preferred_element_type=jnp.float32) + bin_ref[...]
    u = xz[:, :D_INNER]
    z = xz[:, D_INNER:]

    a = jax.nn.sigmoid(alog_ref[...])

    @pl.when(i == 0)
    def _init():
        uprev_ref[...] = jnp.zeros_like(uprev_ref)
        carry_ref[...] = jnp.zeros_like(carry_ref)
        t = lax.broadcasted_iota(jnp.int32, (CHUNK, D_INNER), 0).astype(
            jnp.float32)
        apow_ref[...] = jnp.exp(jnp.log(a) * (t + 1.0))

    ext = jnp.concatenate([uprev_ref[5:8, :], u], axis=0)
    y = (ext[0:CHUNK, :] * cw_ref[0:1, :]
         + ext[1:CHUNK + 1, :] * cw_ref[1:2, :]
         + ext[2:CHUNK + 2, :] * cw_ref[2:3, :]
         + ext[3:CHUNK + 3, :] * cw_ref[3:4, :]) + cb_ref[...]
    uprev_ref[5:8, :] = u[CHUNK - 3:CHUNK, :]
    v = jax.nn.silu(y)

    h = v
    ak = a
    k = 1
    while k < CHUNK:
        top = jnp.zeros((k, D_INNER), jnp.float32)
        h = h + ak * jnp.concatenate([top, h[:CHUNK - k, :]], axis=0)
        ak = ak * ak
        k *= 2
    h = h + apow_ref[...] * carry_ref[...]
    carry_ref[...] = h[CHUNK - 1:CHUNK, :]

    g = h * jax.nn.silu(z)
    m = jnp.dot(g.astype(jnp.bfloat16), wout_ref[...],
                preferred_element_type=jnp.float32) + bout_ref[...]
    o_ref[...] = m
    op_ref[...] = m.astype(jnp.bfloat16)


def _mamba(x, mp):
    win = mp["win"].astype(jnp.bfloat16)
    bin_ = mp["bin"].reshape(1, 2 * D_INNER)
    cw = mp["conv_w"].T
    cb = mp["conv_b"].reshape(1, D_INNER)
    alog = mp["a_logit"].reshape(1, D_INNER)
    wout = mp["wout"].astype(jnp.bfloat16)
    bout = mp["bout"].reshape(1, D)
    n = S // CHUNK
    return pl.pallas_call(
        _mamba_body,
        grid=(n,),
        in_specs=[
            pl.BlockSpec((CHUNK, D), lambda i: (i, 0)),
            pl.BlockSpec((D, 2 * D_INNER), lambda i: (0, 0)),
            pl.BlockSpec((1, 2 * D_INNER), lambda i: (0, 0)),
            pl.BlockSpec((4, D_INNER), lambda i: (0, 0)),
            pl.BlockSpec((1, D_INNER), lambda i: (0, 0)),
            pl.BlockSpec((1, D_INNER), lambda i: (0, 0)),
            pl.BlockSpec((D_INNER, D), lambda i: (0, 0)),
            pl.BlockSpec((1, D), lambda i: (0, 0)),
        ],
        out_specs=[pl.BlockSpec((CHUNK, D), lambda i: (i, 0)),
                   pl.BlockSpec((CHUNK, D), lambda i: (i, 0))],
        out_shape=[jax.ShapeDtypeStruct((S, D), jnp.float32),
                   jax.ShapeDtypeStruct((S, D), jnp.bfloat16)],
        scratch_shapes=[
            pltpu.VMEM((8, D_INNER), jnp.float32),
            pltpu.VMEM((CHUNK, D_INNER), jnp.float32),
            pltpu.VMEM((1, D_INNER), jnp.float32),
        ],
        interpret=False,
    )(x, win, bin_, cw, cb, alog, wout, bout)


def _router_body(ne, x_ref, w_ref, b_ref, wv_ref, ii_ref):
    E = ne + 1
    logits = jnp.dot(x_ref[...], w_ref[...],
                     preferred_element_type=jnp.float32) + b_ref[...]
    lane = lax.broadcasted_iota(jnp.int32, (S, 128), 1)
    valid = lane < E
    lm = jnp.where(valid, logits, -1e30)
    mx = jnp.max(lm, axis=1, keepdims=True)
    e = jnp.where(valid, jnp.exp(lm - mx), 0.0)
    p = e / jnp.sum(e, axis=1, keepdims=True)
    m1 = jnp.max(p, axis=1, keepdims=True)
    i1 = jnp.min(jnp.where(p == m1, lane, 127), axis=1, keepdims=True)
    p2 = jnp.where(lane == i1, -1.0, p)
    m2 = jnp.max(p2, axis=1, keepdims=True)
    i2 = jnp.min(jnp.where(p2 == m2, lane, 127), axis=1, keepdims=True)
    z6 = jnp.zeros((S, 6), jnp.float32)
    wv_ref[...] = jnp.concatenate([m1, m2, z6], axis=1)
    ii_ref[...] = jnp.concatenate(
        [i1, i2, jnp.zeros((S, 6), jnp.int32)], axis=1)


def _router(x, rp, ne):
    E = ne + 1
    w = jnp.pad(rp["w"], ((0, 0), (0, 128 - E)))
    b = jnp.pad(rp["b"], (0, 128 - E)).reshape(1, 128)
    return pl.pallas_call(
        functools.partial(_router_body, ne),
        out_shape=[jax.ShapeDtypeStruct((S, 8), jnp.float32),
                   jax.ShapeDtypeStruct((S, 8), jnp.int32)],
        interpret=False,
    )(x, w, b)


def _rowscan_incl(x):
    n = x.shape[0]
    k = 1
    while k < n:
        top = jnp.zeros((k, x.shape[1]), x.dtype)
        x = x + jnp.concatenate([top, x[:n - k, :]], axis=0)
        k *= 2
    return x


def _lanescan_incl(x):
    n = x.shape[1]
    k = 1
    while k < n:
        x = x + jnp.pad(x, ((0, 0), (k, 0)))[:, :n]
        k *= 2
    return x


def _plan_body(ne, ii_ref, posb_ref, tc_ref):
    lane8 = lax.broadcasted_iota(jnp.int32, (S, 8), 1)
    i0 = ii_ref[:, 0:1]
    i1 = ii_ref[:, 1:2]
    oh0 = ((lane8 == i0) & (lane8 < ne)).astype(jnp.int32)
    oh1 = ((lane8 == i1) & (lane8 < ne)).astype(jnp.int32)
    cum0 = _rowscan_incl(oh0)
    cum1 = _rowscan_incl(oh1)
    tot0 = cum0[S - 1:S, :]
    counts = tot0 + cum1[S - 1:S, :]
    rank0 = jnp.sum(oh0 * (cum0 - oh0), axis=1, keepdims=True)
    rank1 = jnp.sum(oh1 * (cum1 - oh1 + tot0), axis=1, keepdims=True)
    tiles = (counts + (MTILE - 1)) // MTILE
    tincl = _lanescan_incl(tiles)
    pbase = (tincl - tiles) * MTILE
    pb0 = jnp.sum(oh0 * pbase, axis=1, keepdims=True)
    pb1 = jnp.sum(oh1 * pbase, axis=1, keepdims=True)
    dest0 = jnp.where(i0 < ne, pb0 + rank0, NT * MTILE)
    dest1 = jnp.where(i1 < ne, pb1 + rank1, NT * MTILE)
    z6 = jnp.zeros((S, 6), jnp.int32)
    posb_ref[...] = jnp.concatenate([dest0, dest1, z6], axis=1)
    tc_ref[...] = jnp.broadcast_to(tincl, (8, 8))


def _route_plan2(ii, ne):
    posb, tc = pl.pallas_call(
        functools.partial(_plan_body, ne),
        out_shape=[jax.ShapeDtypeStruct((S, 8), jnp.int32),
                   jax.ShapeDtypeStruct((8, 8), jnp.int32)],
        interpret=False,
    )(ii)
    t_ar = jnp.arange(NT, dtype=jnp.int32)
    te = jnp.minimum(jnp.sum(t_ar[:, None] >= tc[0:1, :ne], axis=1),
                     ne - 1).astype(jnp.int32)
    return posb, te


def _expert_body(te_ref, tf_ref, posb_ref, m_ref, w1_ref, b1_ref,
                 w2_ref, b2_ref, y_ref):
    t = pl.program_id(0)
    base = t * MTILE
    rl = base + lax.broadcasted_iota(jnp.int32, (S, MTILE), 1)
    a0 = (posb_ref[:, 0:1] == rl).astype(jnp.bfloat16)
    a1 = (posb_ref[:, 1:2] == rl).astype(jnp.bfloat16)
    m = m_ref[...]
    dn = (((0,), (0,)), ((), ()))
    xg = (lax.dot_general(a0, m, dn, preferred_element_type=jnp.float32)
          + lax.dot_general(a1, m, dn, preferred_element_type=jnp.float32))
    hh = jnp.dot(xg.astype(jnp.bfloat16), w1_ref[0],
                 preferred_element_type=jnp.float32) + b1_ref[0]
    p = hh[:, :DH]
    q = hh[:, DH:]
    flag = tf_ref[te_ref[t]]
    h = jnp.where(flag == 0, jax.nn.silu(p) * q, p * jax.nn.gelu(q))
    y = jnp.dot(h.astype(jnp.bfloat16), w2_ref[0],
                preferred_element_type=jnp.float32) + b2_ref[0]
    y_ref[...] = y.astype(jnp.bfloat16)


def _experts(posb, mb, W1, B1, W2, B2, tflags, te):
    ne = W1.shape[0]
    return pl.pallas_call(
        _expert_body,
        grid_spec=pltpu.PrefetchScalarGridSpec(
            num_scalar_prefetch=2,
            grid=(NT,),
            in_specs=[
                pl.BlockSpec((S, 8), lambda t, te, tf: (0, 0)),
                pl.BlockSpec((S, D), lambda t, te, tf: (0, 0)),
                pl.BlockSpec((1, D, 2 * DH), lambda t, te, tf: (te[t], 0, 0)),
                pl.BlockSpec((1, 1, 2 * DH), lambda t, te, tf: (te[t], 0, 0)),
                pl.BlockSpec((1, DH, D), lambda t, te, tf: (te[t], 0, 0)),
                pl.BlockSpec((1, 1, D), lambda t, te, tf: (te[t], 0, 0)),
            ],
            out_specs=pl.BlockSpec((MTILE, D), lambda t, te, tf: (t, 0)),
        ),
        out_shape=jax.ShapeDtypeStruct((NT * MTILE, D), jnp.bfloat16),
        interpret=False,
    )(te, tflags, posb, mb, W1, B1.reshape(ne, 1, 2 * DH),
      W2, B2.reshape(ne, 1, D))


CCH = 512


def _ln(x, g, b):
    m = jnp.mean(x, axis=-1, keepdims=True)
    v = jnp.mean((x - m) ** 2, axis=-1, keepdims=True)
    return (x - m) / jnp.sqrt(v + 1e-5) * g + b


def _moe_sum(ne, posb_ref, ys_ref, m_ref, wv_ref, ii_ref):
    w0 = wv_ref[:, 0:1]
    w1 = wv_ref[:, 1:2]
    nid0 = (ii_ref[:, 0:1] != ne).astype(jnp.float32)
    nid1 = (ii_ref[:, 1:2] != ne).astype(jnp.float32)
    idw = w0 * (1.0 - nid0) + w1 * (1.0 - nid1)
    rl = lax.broadcasted_iota(jnp.int32, (CCH, NT * MTILE), 1)
    a = (jnp.where(posb_ref[:, 0:1] == rl, w0 * nid0, 0.0)
         + jnp.where(posb_ref[:, 1:2] == rl, w1 * nid1, 0.0))
    g = jnp.dot(a.astype(jnp.bfloat16), ys_ref[...],
                preferred_element_type=jnp.float32)
    return g + idw * m_ref[...]


def _combine_body(ne, posb_ref, ys_ref, m_ref, wv_ref, ii_ref, o_ref):
    o_ref[...] = _moe_sum(ne, posb_ref, ys_ref, m_ref, wv_ref, ii_ref)


def _combine_ln_body(ne, posb_ref, ys_ref, m_ref, wv_ref, ii_ref, res_ref,
                     lg_ref, lb_ref, o_ref):
    moe = _moe_sum(ne, posb_ref, ys_ref, m_ref, wv_ref, ii_ref)
    o_ref[...] = _ln(moe + res_ref[...], lg_ref[...], lb_ref[...])


_CSPECS = [
    pl.BlockSpec((CCH, 8), lambda c: (c, 0)),
    pl.BlockSpec((NT * MTILE, D), lambda c: (0, 0)),
    pl.BlockSpec((CCH, D), lambda c: (c, 0)),
    pl.BlockSpec((CCH, 8), lambda c: (c, 0)),
    pl.BlockSpec((CCH, 8), lambda c: (c, 0)),
]


def _combine(posb, ys, m, wv, ii, ne):
    return pl.pallas_call(
        functools.partial(_combine_body, ne),
        grid=(S // CCH,),
        in_specs=_CSPECS,
        out_specs=pl.BlockSpec((CCH, D), lambda c: (c, 0)),
        out_shape=jax.ShapeDtypeStruct((S, D), jnp.float32),
        interpret=False,
    )(posb, ys, m, wv, ii)


def _combine_ln(posb, ys, m, wv, ii, res, lnp, ne):
    return pl.pallas_call(
        functools.partial(_combine_ln_body, ne),
        grid=(S // CCH,),
        in_specs=_CSPECS + [
            pl.BlockSpec((CCH, D), lambda c: (c, 0)),
            pl.BlockSpec((1, D), lambda c: (0, 0)),
            pl.BlockSpec((1, D), lambda c: (0, 0)),
        ],
        out_specs=pl.BlockSpec((CCH, D), lambda c: (c, 0)),
        out_shape=jax.ShapeDtypeStruct((S, D), jnp.float32),
        interpret=False,
    )(posb, ys, m, wv, ii, res, lnp["g"].reshape(1, D),
      lnp["b"].reshape(1, D))


def _attn_body(x_ref, wq_ref, wk_ref, wv_ref, bq_ref, bk_ref, bv_ref,
               wo_ref, bo_ref, o_ref):
    h = pl.program_id(0)
    x = x_ref[...].astype(jnp.bfloat16)
    q = jnp.dot(x, wq_ref[0], preferred_element_type=jnp.float32) + bq_ref[0]
    k = jnp.dot(x, wk_ref[0], preferred_element_type=jnp.float32) + bk_ref[0]
    v = jnp.dot(x, wv_ref[0], preferred_element_type=jnp.float32) + bv_ref[0]
    s = lax.dot_general(q.astype(jnp.bfloat16), k.astype(jnp.bfloat16),
                        (((1,), (1,)), ((), ())),
                        preferred_element_type=jnp.float32)
    s = s * np.float32(1.0 / np.sqrt(DHEAD))
    mx = jnp.max(s, axis=1, keepdims=True)
    es = jnp.exp(s - mx)
    att = es / jnp.sum(es, axis=1, keepdims=True)
    ov = jnp.dot(att.astype(jnp.bfloat16), v.astype(jnp.bfloat16),
                 preferred_element_type=jnp.float32)
    contrib = jnp.dot(ov.astype(jnp.bfloat16), wo_ref[0],
                      preferred_element_type=jnp.float32)

    @pl.when(h == 0)
    def _init():
        o_ref[...] = jnp.broadcast_to(bo_ref[...], (S, D))

    o_ref[...] += contrib


def _attn(x, ap):
    wqkv = ap["wqkv"]
    bqkv = ap["bqkv"]

    def head_w(wpart):
        w = wpart.reshape(D, N_HEADS, DHEAD).transpose(1, 0, 2)
        return jnp.pad(w, ((0, 0), (0, 0), (0, 128 - DHEAD))).astype(
            jnp.bfloat16)

    def head_b(bpart):
        b = bpart.reshape(N_HEADS, 1, DHEAD)
        return jnp.pad(b, ((0, 0), (0, 0), (0, 128 - DHEAD)))

    wq = head_w(wqkv[:, :D])
    wk = head_w(wqkv[:, D:2 * D])
    wv = head_w(wqkv[:, 2 * D:])
    bq = head_b(bqkv[:D])
    bk = head_b(bqkv[D:2 * D])
    bv = head_b(bqkv[2 * D:])
    wo = jnp.pad(ap["wo"].reshape(N_HEADS, DHEAD, D),
                 ((0, 0), (0, 128 - DHEAD), (0, 0))).astype(jnp.bfloat16)
    bo = ap["bo"].reshape(1, D)
    return pl.pallas_call(
        _attn_body,
        grid=(N_HEADS,),
        in_specs=[
            pl.BlockSpec((S, D), lambda h: (0, 0)),
            pl.BlockSpec((1, D, 128), lambda h: (h, 0, 0)),
            pl.BlockSpec((1, D, 128), lambda h: (h, 0, 0)),
            pl.BlockSpec((1, D, 128), lambda h: (h, 0, 0)),
            pl.BlockSpec((1, 1, 128), lambda h: (h, 0, 0)),
            pl.BlockSpec((1, 1, 128), lambda h: (h, 0, 0)),
            pl.BlockSpec((1, 1, 128), lambda h: (h, 0, 0)),
            pl.BlockSpec((1, 128, D), lambda h: (h, 0, 0)),
            pl.BlockSpec((1, D), lambda h: (0, 0)),
        ],
        out_specs=pl.BlockSpec((S, D), lambda h: (0, 0)),
        out_shape=jax.ShapeDtypeStruct((S, D), jnp.float32),
        interpret=False,
    )(x, wq, wk, wv, bq, bk, bv, wo, bo)


def _postattn_body(xm_ref, ao_ref, res_ref, ga_ref, ba_ref, g_ref, b_ref,
                   o_ref):
    y1 = _ln(xm_ref[...] + ao_ref[...], ga_ref[...], ba_ref[...])
    o_ref[...] = _ln(y1 + res_ref[...], g_ref[...], b_ref[...])


def _postattn(xm, ao, res, lna, lnp):
    return pl.pallas_call(
        _postattn_body,
        out_shape=jax.ShapeDtypeStruct((S, D), jnp.float32),
        interpret=False,
    )(xm, ao, res, lna["g"].reshape(1, D), lna["b"].reshape(1, D),
      lnp["g"].reshape(1, D), lnp["b"].reshape(1, D))


def _lmhead_body(x_ref, w_ref, o_ref):
    o_ref[...] = lax.dot_general(
        x_ref[...].astype(jnp.bfloat16), w_ref[...],
        (((1,), (1,)), ((), ())), preferred_element_type=jnp.float32)


def _lmhead(x, lm):
    V = lm.shape[0]
    VT = 1280
    wb = lm.astype(jnp.bfloat16)
    return pl.pallas_call(
        _lmhead_body,
        grid=(V // VT,),
        in_specs=[
            pl.BlockSpec((S, D), lambda j: (0, 0)),
            pl.BlockSpec((VT, D), lambda j: (j, 0)),
        ],
        out_specs=pl.BlockSpec((S, VT), lambda j: (0, j)),
        out_shape=jax.ShapeDtypeStruct((S, V), jnp.float32),
        interpret=False,
    )(x, wb)


_ETYPES = [["conv"] * 4 + ["geglu"] * 2 + ["id"],
           ["conv"] * 3 + ["geglu"] * 3 + ["id"],
           ["conv"] * 1 + ["geglu"] * 4 + ["id"]]
_NBLOCKS = [1, 2, 1]
_USE_ATTN = [False, False, True]


def _stack_experts(experts, etypes):
    w1s, b1s, w2s, b2s, flags = [], [], [], [], []
    for et, ep in zip(etypes, experts):
        if et == "conv":
            w1s.append(jnp.concatenate([ep["w1"], ep["w3"]], axis=1))
            b1s.append(jnp.concatenate([ep["b1"], ep["b3"]]))
            flags.append(0)
        elif et == "geglu":
            w1s.append(ep["w1"])
            b1s.append(ep["b1"])
            flags.append(1)
        else:
            continue
        w2s.append(ep["w2"])
        b2s.append(ep["b2"])
    return (jnp.stack(w1s).astype(jnp.bfloat16), jnp.stack(b1s),
            jnp.stack(w2s).astype(jnp.bfloat16), jnp.stack(b2s),
            jnp.asarray(flags, jnp.int32))


def kernel(input_ids, params):
    ids = input_ids.reshape(S).astype(jnp.int32)
    x = _gather_rows(params["embed"], ids)
    for s in range(3):
        st = params["stages"][s]
        etypes = _ETYPES[s]
        ne = len(etypes) - 1
        W1, B1, W2, B2, tflags = _stack_experts(st["experts"], etypes)
        for bi in range(_NBLOCKS[s]):
            bp = st["blocks"][bi]
            res = x
            m, mb = _mamba(x, bp["mamba"])
            wv, ii = _router(m, bp["router"], ne)
            posb, te = _route_plan2(ii, ne)
            ys = _experts(posb, mb, W1, B1, W2, B2, tflags, te)
            if _USE_ATTN[s]:
                xm = _combine(posb, ys, m, wv, ii, ne)
                ao = _attn(xm, bp["attn"])
                x = _postattn(xm, ao, res, bp["ln_attn"], bp["ln"])
            else:
                x = _combine_ln(posb, ys, m, wv, ii, res, bp["ln"], ne)
    logits = _lmhead(x, params["lm_head"])
    return logits.reshape(1, S, -1)
